# baseline (speedup 1.0000x reference)
import functools

import jax
import jax.numpy as jnp
from jax import lax
from jax.experimental import pallas as pl
from jax.experimental.pallas import tpu as pltpu
from jax.experimental.pallas import tpu_sc as plsc

_N = 10000
_D = 128
_E = 320000

_NC = 2
_NS = 16
_NW = _NC * _NS
_CH = 128
_NCH = (-(-_E // (_NW * _CH)) + 7) // 8 * 8
_EPAD = _NW * _CH * _NCH
_K = 16
_C0 = 80
_C1 = 80
_PST = _C0 + _C1
_NP = 10240
_RPS = _NP // _NS
_RB = 2000


def _deg_body(dst_hbm, out_hbm, idx_d, ones_v, zb, deg, dsem):
    cid = lax.axis_index("c")
    sid = lax.axis_index("s")
    wid = sid * _NC + cid

    def _z16(i, c):
        zb[pl.ds(i * 16, 16)] = jnp.zeros((16,), jnp.float32)
        return c

    lax.fori_loop(0, _RPS // 16, _z16, 0)
    for j in range(_CH // 16):
        ones_v[pl.ds(j * 16, 16)] = jnp.ones((16,), jnp.float32)

    base = sid * _RPS
    pltpu.sync_copy(zb, deg.at[pl.ds(base, _RPS)])
    pltpu.sync_copy(dst_hbm.at[pl.ds(wid * _NCH, _NCH)], idx_d)
    plsc.subcore_barrier()

    hs = []
    for g in range(_NCH):
        hs.append(pltpu.async_copy(ones_v, deg.at[idx_d.at[g]], dsem, add=True))
    for h in hs:
        h.wait()

    plsc.subcore_barrier()
    pltpu.sync_copy(deg.at[pl.ds(base, _RPS)],
                    out_hbm.at[pl.ds(cid * _NP + base, _RPS)])


_NBUF = 8
_GA = 6
_CW = 32
_KI = 32
_C0W = 512
_C1W = 128
_PSTW = _C0W + _C1W


def _conv_body(ts_hbm, src_hbm, dst_hbm, out_hbm,
               idx_s, idx_d, *bufs):
    rows = bufs[:_NBUF]
    acc = bufs[_NBUF]
    gsem = bufs[_NBUF + 1:2 * _NBUF + 1]
    ssem = bufs[2 * _NBUF + 1:]
    r0 = rows[0]
    cid = lax.axis_index("c")
    sid = lax.axis_index("s")

    def _zrow(i, c):
        for j in range(_D // 16):
            r0[i, pl.ds(j * 16, 16)] = jnp.zeros((16,), jnp.float32)
        return c

    lax.fori_loop(0, _CW, _zrow, 0)

    base = sid * _RPS
    for r in range(_RPS // _CW):
        pltpu.sync_copy(r0, acc.at[pl.ds(base + r * _CW, _CW)])
    plsc.subcore_barrier()

    pair = sid * _PSTW
    row0 = jnp.where(cid == 0, pair, pair + _C0W)
    nsc = jnp.where(cid == 0, _C0W // _KI, _C1W // _KI)

    def _super(j, c):
        hb = pl.multiple_of(row0 + j * _KI, 8)
        pltpu.sync_copy(src_hbm.at[pl.ds(hb, _KI)], idx_s)
        pltpu.sync_copy(dst_hbm.at[pl.ds(hb, _KI)], idx_d)
        gh = [None] * _NBUF
        sh = [None] * _NBUF
        for g in range(_GA):
            gh[g] = pltpu.async_copy(ts_hbm.at[idx_s.at[g]], rows[g],
                                     gsem[g])
        for g in range(_KI):
            b = g % _NBUF
            gh[b].wait()
            sh[b] = pltpu.async_copy(rows[b], acc.at[idx_d.at[g]],
                                     ssem[b], add=True)
            ng = g + _GA
            if ng < _KI:
                nb = ng % _NBUF
                if sh[nb] is not None:
                    sh[nb].wait()
                    sh[nb] = None
                gh[nb] = pltpu.async_copy(ts_hbm.at[idx_s.at[ng]],
                                          rows[nb], gsem[nb])
        for b in range(_NBUF):
            if sh[b] is not None:
                sh[b].wait()
        return c

    lax.fori_loop(0, nsc, _super, 0)

    plsc.subcore_barrier()
    pltpu.sync_copy(acc.at[pl.ds(base, _RPS)],
                    out_hbm.at[pl.ds(cid * _NP + base, _RPS)])


@functools.cache
def _sc_calls():
    mesh = plsc.VectorSubcoreMesh(core_axis_name="c", subcore_axis_name="s")
    deg_call = pl.kernel(
        _deg_body,
        out_type=jax.ShapeDtypeStruct((_NC * _NP,), jnp.float32),
        mesh=mesh,
        scratch_types=[
            pltpu.VMEM((_NCH, _CH), jnp.int32),
            pltpu.VMEM((_CH,), jnp.float32),
            pltpu.VMEM((_RPS,), jnp.float32),
            pltpu.VMEM_SHARED((_NP,), jnp.float32),
            pltpu.SemaphoreType.DMA,
        ],
    )
    conv_call = pl.kernel(
        _conv_body,
        out_type=jax.ShapeDtypeStruct((_NC * _NP, _D), jnp.float32),
        mesh=mesh,
        scratch_types=[
            pltpu.VMEM((_KI, _CW), jnp.int32),
            pltpu.VMEM((_KI, _CW), jnp.int32),
        ] + [pltpu.VMEM((_CW, _D), jnp.float32)] * _NBUF + [
            pltpu.VMEM_SHARED((_NP, _D), jnp.float32),
        ] + [pltpu.SemaphoreType.DMA] * (2 * _NBUF),
    )
    return deg_call, conv_call


def _k1_body(x_ref, win_ref, bin_ref, wc1_ref, d0_ref, d1_ref,
             ts_ref, dis_ref):
    h0 = jnp.maximum(
        jnp.dot(x_ref[...], win_ref[...], preferred_element_type=jnp.float32)
        + bin_ref[...], 0.0)
    deg = d0_ref[0] + d1_ref[0] + 1.0
    dis = lax.rsqrt(deg)
    dis_ref[...] = dis
    ts_ref[...] = jnp.dot(h0, wc1_ref[...],
                          preferred_element_type=jnp.float32) * dis


def _mid_body(a0_ref, a1_ref, ts_ref, dis_ref, bc_ref, g_ref, be_ref, w_ref,
              o_ref):
    dis = dis_ref[...]
    u = (a0_ref[0] + a1_ref[0] + ts_ref[...]) * dis + bc_ref[...]
    m = jnp.mean(u, axis=-1, keepdims=True)
    c = u - m
    v = jnp.mean(c * c, axis=-1, keepdims=True)
    h = jnp.maximum(c * lax.rsqrt(v + 1e-5) * g_ref[...] + be_ref[...], 0.0)
    o_ref[...] = jnp.dot(h, w_ref[...],
                         preferred_element_type=jnp.float32) * dis


def _fin_body(a0_ref, a1_ref, ts_ref, dis_ref, bc_ref, g_ref, be_ref, w_ref,
              bo_ref, o_ref):
    u = (a0_ref[0] + a1_ref[0] + ts_ref[...]) * dis_ref[...] + bc_ref[...]
    m = jnp.mean(u, axis=-1, keepdims=True)
    c = u - m
    v = jnp.mean(c * c, axis=-1, keepdims=True)
    h = jnp.maximum(c * lax.rsqrt(v + 1e-5) * g_ref[...] + be_ref[...], 0.0)
    o_ref[...] = jnp.dot(h, w_ref[...],
                         preferred_element_type=jnp.float32) + bo_ref[...]


_GRID = (_N // _RB,)

def _row_spec():
    return pl.BlockSpec((_RB, _D), lambda i: (i, 0))

def _w_spec():
    return pl.BlockSpec((_D, _D), lambda i: (0, 0))

def _b_spec():
    return pl.BlockSpec((1, _D), lambda i: (0, 0))

def _part_spec(c):
    return pl.BlockSpec((1, _RB, _D), lambda i, c=c: (c, i, 0))

def _deg_spec(c):
    return pl.BlockSpec((1, _RB, 1), lambda i, c=c: (c, i, 0))

def _dis_spec():
    return pl.BlockSpec((_RB, 1), lambda i: (i, 0))


_k1_call = pl.pallas_call(
    _k1_body,
    grid=_GRID,
    in_specs=[_row_spec(), _w_spec(), _b_spec(), _w_spec(),
              _deg_spec(0), _deg_spec(1)],
    out_specs=[_row_spec(), _dis_spec()],
    out_shape=[jax.ShapeDtypeStruct((_N, _D), jnp.float32),
               jax.ShapeDtypeStruct((_N, 1), jnp.float32)],
)

_mid_call = pl.pallas_call(
    _mid_body,
    grid=_GRID,
    in_specs=[_part_spec(0), _part_spec(1), _row_spec(), _dis_spec(),
              _b_spec(), _b_spec(), _b_spec(), _w_spec()],
    out_specs=_row_spec(),
    out_shape=jax.ShapeDtypeStruct((_N, _D), jnp.float32),
)

_fin_call = pl.pallas_call(
    _fin_body,
    grid=_GRID,
    in_specs=[_part_spec(0), _part_spec(1), _row_spec(), _dis_spec(),
              _b_spec(), _b_spec(), _b_spec(), _w_spec(), _b_spec()],
    out_specs=_row_spec(),
    out_shape=jax.ShapeDtypeStruct((_N, _D), jnp.float32),
)


def kernel(x, edge_index, W_in, b_in, W_c1, b_c1, g1, be1,
           W_c2, b_c2, g2, be2, W_out, b_out):
    src = edge_index[0].astype(jnp.int32)
    dst = edge_index[1].astype(jnp.int32)
    pad = _EPAD - _E
    srcf = jnp.concatenate([src, jnp.zeros((pad,), jnp.int32)])
    dstf = jnp.concatenate([dst, jnp.full((pad,), _N, jnp.int32)])
    srcp = srcf.reshape(_EPAD // _CW, _CW)
    dstp = dstf.reshape(_EPAD // _CW, _CW)
    dstp128 = dstf.reshape(_EPAD // _CH, _CH)

    deg_call, conv_call = _sc_calls()
    degp = deg_call(dstp128).reshape(_NC, _NP, 1)
    ts1, dis = _k1_call(x, W_in, b_in.reshape(1, _D), W_c1, degp, degp)
    acc1 = conv_call(ts1, srcp, dstp).reshape(_NC, _NP, _D)
    ts2 = _mid_call(acc1, acc1, ts1, dis, b_c1.reshape(1, _D),
                    g1.reshape(1, _D), be1.reshape(1, _D), W_c2)
    acc2 = conv_call(ts2, srcp, dstp).reshape(_NC, _NP, _D)
    return _fin_call(acc2, acc2, ts2, dis, b_c2.reshape(1, _D),
                     g2.reshape(1, _D), be2.reshape(1, _D), W_out,
                     b_out.reshape(1, _D))


# speedup vs baseline: 1.0382x; 1.0382x over previous
import functools

import jax
import jax.numpy as jnp
from jax import lax
from jax.experimental import pallas as pl
from jax.experimental.pallas import tpu as pltpu
from jax.experimental.pallas import tpu_sc as plsc

_N = 10000
_D = 128
_E = 320000

_NC = 2
_NS = 16
_NW = _NC * _NS
_CH = 128
_NCH = (-(-_E // (_NW * _CH)) + 7) // 8 * 8
_EPAD = _NW * _CH * _NCH
_K = 16
_C0 = 80
_C1 = 80
_PST = _C0 + _C1
_NP = 10240
_RPS = _NP // _NS
_RB = 2000


def _deg_body(dst_hbm, out_hbm, idx_d, ones_v, zb, deg, dsem):
    cid = lax.axis_index("c")
    sid = lax.axis_index("s")
    wid = sid * _NC + cid

    def _z16(i, c):
        zb[pl.ds(i * 16, 16)] = jnp.zeros((16,), jnp.float32)
        return c

    lax.fori_loop(0, _RPS // 16, _z16, 0)
    for j in range(_CH // 16):
        ones_v[pl.ds(j * 16, 16)] = jnp.ones((16,), jnp.float32)

    base = sid * _RPS
    pltpu.sync_copy(zb, deg.at[pl.ds(base, _RPS)])
    pltpu.sync_copy(dst_hbm.at[pl.ds(wid * _NCH, _NCH)], idx_d)
    plsc.subcore_barrier()

    hs = []
    for g in range(_NCH):
        hs.append(pltpu.async_copy(ones_v, deg.at[idx_d.at[g]], dsem, add=True))
    for h in hs:
        h.wait()

    plsc.subcore_barrier()
    pltpu.sync_copy(deg.at[pl.ds(base, _RPS)],
                    out_hbm.at[pl.ds(cid * _NP + base, _RPS)])


_NBUF = 5
_GA = 3
_CW = 64
_KI = 16
_C0W = 240
_C1W = 80
_PSTW = _C0W + _C1W


def _conv_body(ts_hbm, src_hbm, dst_hbm, out_hbm,
               idx_s, idx_d, r0, r1, r2, r3, r4, acc,
               g0, g1, g2, g3, g4, s0, s1, s2, s3, s4):
    cid = lax.axis_index("c")
    sid = lax.axis_index("s")

    def _zrow(i, c):
        for j in range(_D // 16):
            r0[i, pl.ds(j * 16, 16)] = jnp.zeros((16,), jnp.float32)
        return c

    lax.fori_loop(0, _CW, _zrow, 0)

    base = sid * _RPS
    for r in range(_RPS // _CW):
        pltpu.sync_copy(r0, acc.at[pl.ds(base + r * _CW, _CW)])
    plsc.subcore_barrier()

    rows = (r0, r1, r2, r3, r4)
    gsem = (g0, g1, g2, g3, g4)
    ssem = (s0, s1, s2, s3, s4)

    pair = sid * _PSTW
    row0 = jnp.where(cid == 0, pair, pair + _C0W)
    nsc = jnp.where(cid == 0, _C0W // _KI, _C1W // _KI)

    def _super(j, c):
        hb = pl.multiple_of(row0 + j * _KI, 8)
        pltpu.sync_copy(src_hbm.at[pl.ds(hb, _KI)], idx_s)
        pltpu.sync_copy(dst_hbm.at[pl.ds(hb, _KI)], idx_d)
        gh = [None] * _NBUF
        sh = [None] * _NBUF
        for g in range(_GA):
            gh[g] = pltpu.async_copy(ts_hbm.at[idx_s.at[g]], rows[g],
                                     gsem[g])
        for g in range(_KI):
            b = g % _NBUF
            gh[b].wait()
            sh[b] = pltpu.async_copy(rows[b], acc.at[idx_d.at[g]],
                                     ssem[b], add=True)
            ng = g + _GA
            if ng < _KI:
                nb = ng % _NBUF
                if sh[nb] is not None:
                    sh[nb].wait()
                    sh[nb] = None
                gh[nb] = pltpu.async_copy(ts_hbm.at[idx_s.at[ng]],
                                          rows[nb], gsem[nb])
        for b in range(_NBUF):
            if sh[b] is not None:
                sh[b].wait()
        return c

    lax.fori_loop(0, nsc, _super, 0)

    plsc.subcore_barrier()
    pltpu.sync_copy(acc.at[pl.ds(base, _RPS)],
                    out_hbm.at[pl.ds(cid * _NP + base, _RPS)])


@functools.cache
def _sc_calls():
    mesh = plsc.VectorSubcoreMesh(core_axis_name="c", subcore_axis_name="s")
    deg_call = pl.kernel(
        _deg_body,
        out_type=jax.ShapeDtypeStruct((_NC * _NP,), jnp.float32),
        mesh=mesh,
        scratch_types=[
            pltpu.VMEM((_NCH, _CH), jnp.int32),
            pltpu.VMEM((_CH,), jnp.float32),
            pltpu.VMEM((_RPS,), jnp.float32),
            pltpu.VMEM_SHARED((_NP,), jnp.float32),
            pltpu.SemaphoreType.DMA,
        ],
    )
    conv_call = pl.kernel(
        _conv_body,
        out_type=jax.ShapeDtypeStruct((_NC * _NP, _D), jnp.float32),
        mesh=mesh,
        scratch_types=[
            pltpu.VMEM((_KI, _CW), jnp.int32),
            pltpu.VMEM((_KI, _CW), jnp.int32),
            pltpu.VMEM((_CW, _D), jnp.float32),
            pltpu.VMEM((_CW, _D), jnp.float32),
            pltpu.VMEM((_CW, _D), jnp.float32),
            pltpu.VMEM((_CW, _D), jnp.float32),
            pltpu.VMEM((_CW, _D), jnp.float32),
            pltpu.VMEM_SHARED((_NP, _D), jnp.float32),
        ] + [pltpu.SemaphoreType.DMA] * (2 * _NBUF),
    )
    return deg_call, conv_call


def _k1_body(x_ref, win_ref, bin_ref, wc1_ref, d0_ref, d1_ref,
             ts_ref, dis_ref):
    h0 = jnp.maximum(
        jnp.dot(x_ref[...], win_ref[...], preferred_element_type=jnp.float32)
        + bin_ref[...], 0.0)
    deg = d0_ref[0] + d1_ref[0] + 1.0
    dis = lax.rsqrt(deg)
    dis_ref[...] = dis
    ts_ref[...] = jnp.dot(h0, wc1_ref[...],
                          preferred_element_type=jnp.float32) * dis


def _mid_body(a0_ref, a1_ref, ts_ref, dis_ref, bc_ref, g_ref, be_ref, w_ref,
              o_ref):
    dis = dis_ref[...]
    u = (a0_ref[0] + a1_ref[0] + ts_ref[...]) * dis + bc_ref[...]
    m = jnp.mean(u, axis=-1, keepdims=True)
    c = u - m
    v = jnp.mean(c * c, axis=-1, keepdims=True)
    h = jnp.maximum(c * lax.rsqrt(v + 1e-5) * g_ref[...] + be_ref[...], 0.0)
    o_ref[...] = jnp.dot(h, w_ref[...],
                         preferred_element_type=jnp.float32) * dis


def _fin_body(a0_ref, a1_ref, ts_ref, dis_ref, bc_ref, g_ref, be_ref, w_ref,
              bo_ref, o_ref):
    u = (a0_ref[0] + a1_ref[0] + ts_ref[...]) * dis_ref[...] + bc_ref[...]
    m = jnp.mean(u, axis=-1, keepdims=True)
    c = u - m
    v = jnp.mean(c * c, axis=-1, keepdims=True)
    h = jnp.maximum(c * lax.rsqrt(v + 1e-5) * g_ref[...] + be_ref[...], 0.0)
    o_ref[...] = jnp.dot(h, w_ref[...],
                         preferred_element_type=jnp.float32) + bo_ref[...]


_GRID = (_N // _RB,)

def _row_spec():
    return pl.BlockSpec((_RB, _D), lambda i: (i, 0))

def _w_spec():
    return pl.BlockSpec((_D, _D), lambda i: (0, 0))

def _b_spec():
    return pl.BlockSpec((1, _D), lambda i: (0, 0))

def _part_spec(c):
    return pl.BlockSpec((1, _RB, _D), lambda i, c=c: (c, i, 0))

def _deg_spec(c):
    return pl.BlockSpec((1, _RB, 1), lambda i, c=c: (c, i, 0))

def _dis_spec():
    return pl.BlockSpec((_RB, 1), lambda i: (i, 0))


_k1_call = pl.pallas_call(
    _k1_body,
    grid=_GRID,
    in_specs=[_row_spec(), _w_spec(), _b_spec(), _w_spec(),
              _deg_spec(0), _deg_spec(1)],
    out_specs=[_row_spec(), _dis_spec()],
    out_shape=[jax.ShapeDtypeStruct((_N, _D), jnp.float32),
               jax.ShapeDtypeStruct((_N, 1), jnp.float32)],
)

_mid_call = pl.pallas_call(
    _mid_body,
    grid=_GRID,
    in_specs=[_part_spec(0), _part_spec(1), _row_spec(), _dis_spec(),
              _b_spec(), _b_spec(), _b_spec(), _w_spec()],
    out_specs=_row_spec(),
    out_shape=jax.ShapeDtypeStruct((_N, _D), jnp.float32),
)

_fin_call = pl.pallas_call(
    _fin_body,
    grid=_GRID,
    in_specs=[_part_spec(0), _part_spec(1), _row_spec(), _dis_spec(),
              _b_spec(), _b_spec(), _b_spec(), _w_spec(), _b_spec()],
    out_specs=_row_spec(),
    out_shape=jax.ShapeDtypeStruct((_N, _D), jnp.float32),
)


def kernel(x, edge_index, W_in, b_in, W_c1, b_c1, g1, be1,
           W_c2, b_c2, g2, be2, W_out, b_out):
    src = edge_index[0].astype(jnp.int32)
    dst = edge_index[1].astype(jnp.int32)
    pad = _EPAD - _E
    srcf = jnp.concatenate([src, jnp.zeros((pad,), jnp.int32)])
    dstf = jnp.concatenate([dst, jnp.full((pad,), _N, jnp.int32)])
    srcp = srcf.reshape(_EPAD // _CW, _CW)
    dstp = dstf.reshape(_EPAD // _CW, _CW)
    dstp128 = dstf.reshape(_EPAD // _CH, _CH)

    deg_call, conv_call = _sc_calls()
    degp = deg_call(dstp128).reshape(_NC, _NP, 1)
    ts1, dis = _k1_call(x, W_in, b_in.reshape(1, _D), W_c1, degp, degp)
    acc1 = conv_call(ts1, srcp, dstp).reshape(_NC, _NP, _D)
    ts2 = _mid_call(acc1, acc1, ts1, dis, b_c1.reshape(1, _D),
                    g1.reshape(1, _D), be1.reshape(1, _D), W_c2)
    acc2 = conv_call(ts2, srcp, dstp).reshape(_NC, _NP, _D)
    return _fin_call(acc2, acc2, ts2, dis, b_c2.reshape(1, _D),
                     g2.reshape(1, _D), be2.reshape(1, _D), W_out,
                     b_out.reshape(1, _D))


# speedup vs baseline: 1.0384x; 1.0002x over previous
import functools

import jax
import jax.numpy as jnp
from jax import lax
from jax.experimental import pallas as pl
from jax.experimental.pallas import tpu as pltpu
from jax.experimental.pallas import tpu_sc as plsc

_N = 10000
_D = 128
_E = 320000

_NC = 2
_NS = 16
_NW = _NC * _NS
_CH = 128
_NCH = (-(-_E // (_NW * _CH)) + 7) // 8 * 8
_EPAD = _NW * _CH * _NCH
_NP = 10240
_RPS = _NP // _NS
_RB = 2000


def _deg_body(dst_hbm, out_hbm, idx_d, ones_v, zb, deg, dsem):
    cid = lax.axis_index("c")
    sid = lax.axis_index("s")
    wid = sid * _NC + cid

    def _z16(i, c):
        zb[pl.ds(i * 16, 16)] = jnp.zeros((16,), jnp.float32)
        return c

    lax.fori_loop(0, _RPS // 16, _z16, 0)
    for j in range(_CH // 16):
        ones_v[pl.ds(j * 16, 16)] = jnp.ones((16,), jnp.float32)

    base = sid * _RPS
    pltpu.sync_copy(zb, deg.at[pl.ds(base, _RPS)])
    pltpu.sync_copy(dst_hbm.at[pl.ds(wid * _NCH, _NCH)], idx_d)
    plsc.subcore_barrier()

    hs = []
    for g in range(_NCH):
        hs.append(pltpu.async_copy(ones_v, deg.at[idx_d.at[g]], dsem, add=True))
    for h in hs:
        h.wait()

    plsc.subcore_barrier()
    pltpu.sync_copy(deg.at[pl.ds(base, _RPS)],
                    out_hbm.at[pl.ds(cid * _NP + base, _RPS)])


_NBUF = 5
_GA = 3
_CW = 64
_KI = 16
_C0W = 240
_C1W = 80
_PSTW = _C0W + _C1W


def _conv_body(ts_hbm, src_hbm, dst_hbm, out_hbm,
               idx_s, idx_d, r0, r1, r2, r3, r4, acc,
               g0, g1, g2, g3, g4, s0, s1, s2, s3, s4):
    cid = lax.axis_index("c")
    sid = lax.axis_index("s")

    def _zrow(i, c):
        for j in range(_D // 16):
            r0[i, pl.ds(j * 16, 16)] = jnp.zeros((16,), jnp.float32)
        return c

    lax.fori_loop(0, _CW, _zrow, 0)

    base = sid * _RPS
    for r in range(_RPS // _CW):
        pltpu.sync_copy(r0, acc.at[pl.ds(base + r * _CW, _CW)])
    plsc.subcore_barrier()

    rows = (r0, r1, r2, r3, r4)
    gsem = (g0, g1, g2, g3, g4)
    ssem = (s0, s1, s2, s3, s4)

    pair = sid * _PSTW
    row0 = jnp.where(cid == 0, pair, pair + _C0W)
    nsc = jnp.where(cid == 0, _C0W // _KI, _C1W // _KI)

    def _super(j, c):
        hb = pl.multiple_of(row0 + j * _KI, 8)
        pltpu.sync_copy(src_hbm.at[pl.ds(hb, _KI)], idx_s)
        pltpu.sync_copy(dst_hbm.at[pl.ds(hb, _KI)], idx_d)
        gh = [None] * _NBUF
        sh = [None] * _NBUF
        for g in range(_GA):
            gh[g] = pltpu.async_copy(ts_hbm.at[idx_s.at[g]], rows[g],
                                     gsem[g])
        for g in range(_KI):
            b = g % _NBUF
            gh[b].wait()
            sh[b] = pltpu.async_copy(rows[b], acc.at[idx_d.at[g]],
                                     ssem[b], add=True)
            ng = g + _GA
            if ng < _KI:
                nb = ng % _NBUF
                if sh[nb] is not None:
                    sh[nb].wait()
                    sh[nb] = None
                gh[nb] = pltpu.async_copy(ts_hbm.at[idx_s.at[ng]],
                                          rows[nb], gsem[nb])
        for b in range(_NBUF):
            if sh[b] is not None:
                sh[b].wait()
        return c

    lax.fori_loop(0, nsc, _super, 0)

    plsc.subcore_barrier()
    pltpu.sync_copy(acc.at[pl.ds(base, _RPS)],
                    out_hbm.at[pl.ds(cid * _NP + base, _RPS)])


@functools.cache
def _sc_calls():
    mesh = plsc.VectorSubcoreMesh(core_axis_name="c", subcore_axis_name="s")
    deg_call = pl.kernel(
        _deg_body,
        out_type=jax.ShapeDtypeStruct((_NC * _NP,), jnp.float32),
        mesh=mesh,
        scratch_types=[
            pltpu.VMEM((_NCH, _CH), jnp.int32),
            pltpu.VMEM((_CH,), jnp.float32),
            pltpu.VMEM((_RPS,), jnp.float32),
            pltpu.VMEM_SHARED((_NP,), jnp.float32),
            pltpu.SemaphoreType.DMA,
        ],
    )
    conv_call = pl.kernel(
        _conv_body,
        out_type=jax.ShapeDtypeStruct((_NC * _NP, _D), jnp.float32),
        mesh=mesh,
        scratch_types=[
            pltpu.VMEM((_KI, _CW), jnp.int32),
            pltpu.VMEM((_KI, _CW), jnp.int32),
            pltpu.VMEM((_CW, _D), jnp.float32),
            pltpu.VMEM((_CW, _D), jnp.float32),
            pltpu.VMEM((_CW, _D), jnp.float32),
            pltpu.VMEM((_CW, _D), jnp.float32),
            pltpu.VMEM((_CW, _D), jnp.float32),
            pltpu.VMEM_SHARED((_NP, _D), jnp.float32),
        ] + [pltpu.SemaphoreType.DMA] * (2 * _NBUF),
    )
    return deg_call, conv_call


def _k1_body(x_ref, win_ref, bin_ref, wc1_ref, d0_ref, d1_ref,
             ts_ref, dis_ref):
    h0 = jnp.maximum(
        jnp.dot(x_ref[...], win_ref[...], preferred_element_type=jnp.float32)
        + bin_ref[...], 0.0)
    deg = d0_ref[0] + d1_ref[0] + 1.0
    dis = lax.rsqrt(deg)
    dis_ref[...] = dis
    ts_ref[...] = jnp.dot(h0, wc1_ref[...],
                          preferred_element_type=jnp.float32) * dis


def _mid_body(a0_ref, a1_ref, ts_ref, dis_ref, bc_ref, g_ref, be_ref, w_ref,
              o_ref):
    dis = dis_ref[...]
    u = (a0_ref[0] + a1_ref[0] + ts_ref[...]) * dis + bc_ref[...]
    m = jnp.mean(u, axis=-1, keepdims=True)
    c = u - m
    v = jnp.mean(c * c, axis=-1, keepdims=True)
    h = jnp.maximum(c * lax.rsqrt(v + 1e-5) * g_ref[...] + be_ref[...], 0.0)
    o_ref[...] = jnp.dot(h, w_ref[...],
                         preferred_element_type=jnp.float32) * dis


def _fin_body(a0_ref, a1_ref, ts_ref, dis_ref, bc_ref, g_ref, be_ref, w_ref,
              bo_ref, o_ref):
    u = (a0_ref[0] + a1_ref[0] + ts_ref[...]) * dis_ref[...] + bc_ref[...]
    m = jnp.mean(u, axis=-1, keepdims=True)
    c = u - m
    v = jnp.mean(c * c, axis=-1, keepdims=True)
    h = jnp.maximum(c * lax.rsqrt(v + 1e-5) * g_ref[...] + be_ref[...], 0.0)
    o_ref[...] = jnp.dot(h, w_ref[...],
                         preferred_element_type=jnp.float32) + bo_ref[...]


_GRID = (_N // _RB,)

def _row_spec():
    return pl.BlockSpec((_RB, _D), lambda i: (i, 0))

def _w_spec():
    return pl.BlockSpec((_D, _D), lambda i: (0, 0))

def _b_spec():
    return pl.BlockSpec((1, _D), lambda i: (0, 0))

def _part_spec(c):
    return pl.BlockSpec((1, _RB, _D), lambda i, c=c: (c, i, 0))

def _deg_spec(c):
    return pl.BlockSpec((1, _RB, 1), lambda i, c=c: (c, i, 0))

def _dis_spec():
    return pl.BlockSpec((_RB, 1), lambda i: (i, 0))


_k1_call = pl.pallas_call(
    _k1_body,
    grid=_GRID,
    in_specs=[_row_spec(), _w_spec(), _b_spec(), _w_spec(),
              _deg_spec(0), _deg_spec(1)],
    out_specs=[_row_spec(), _dis_spec()],
    out_shape=[jax.ShapeDtypeStruct((_N, _D), jnp.float32),
               jax.ShapeDtypeStruct((_N, 1), jnp.float32)],
)

_mid_call = pl.pallas_call(
    _mid_body,
    grid=_GRID,
    in_specs=[_part_spec(0), _part_spec(1), _row_spec(), _dis_spec(),
              _b_spec(), _b_spec(), _b_spec(), _w_spec()],
    out_specs=_row_spec(),
    out_shape=jax.ShapeDtypeStruct((_N, _D), jnp.float32),
)

_fin_call = pl.pallas_call(
    _fin_body,
    grid=_GRID,
    in_specs=[_part_spec(0), _part_spec(1), _row_spec(), _dis_spec(),
              _b_spec(), _b_spec(), _b_spec(), _w_spec(), _b_spec()],
    out_specs=_row_spec(),
    out_shape=jax.ShapeDtypeStruct((_N, _D), jnp.float32),
)


def kernel(x, edge_index, W_in, b_in, W_c1, b_c1, g1, be1,
           W_c2, b_c2, g2, be2, W_out, b_out):
    src = edge_index[0].astype(jnp.int32)
    dst = edge_index[1].astype(jnp.int32)
    pad = _EPAD - _E
    srcf = jnp.concatenate([src, jnp.zeros((pad,), jnp.int32)])
    dstf = jnp.concatenate([dst, jnp.full((pad,), _N, jnp.int32)])
    srcp = srcf.reshape(_EPAD // _CW, _CW)
    dstp = dstf.reshape(_EPAD // _CW, _CW)
    dstp128 = dstf.reshape(_EPAD // _CH, _CH)

    deg_call, conv_call = _sc_calls()
    degp = deg_call(dstp128).reshape(_NC, _NP, 1)
    ts1, dis = _k1_call(x, W_in, b_in.reshape(1, _D), W_c1, degp, degp)
    acc1 = conv_call(ts1, srcp, dstp).reshape(_NC, _NP, _D)
    ts2 = _mid_call(acc1, acc1, ts1, dis, b_c1.reshape(1, _D),
                    g1.reshape(1, _D), be1.reshape(1, _D), W_c2)
    acc2 = conv_call(ts2, srcp, dstp).reshape(_NC, _NP, _D)
    return _fin_call(acc2, acc2, ts2, dis, b_c2.reshape(1, _D),
                     g2.reshape(1, _D), be2.reshape(1, _D), W_out,
                     b_out.reshape(1, _D))
